# Initial kernel scaffold; baseline (speedup 1.0000x reference)
import jax
import jax.numpy as jnp
from jax.experimental import pallas as pl


def kernel(state, edge_index, deterministic, W1, b1, W2, b2, W3, b3, lw1, lb1, lw2, lb2, lw3, lb3):
    raise NotImplementedError("write your pallas kernel here")


# speedup vs baseline: 19.0073x; 19.0073x over previous
import functools

import jax
import jax.numpy as jnp
from jax import lax
from jax.experimental import pallas as pl
from jax.experimental.pallas import tpu as pltpu
from jax.experimental.pallas import tpu_sc as plsc

N = 10000
NPAD = 10240
E = 320000
NC = 2
NS = 16
NW = NC * NS
EPW = 10240
CHUNK = 128
NCH = EPW // CHUNK
RPT = NPAD // NS
PAD_NODE = N + 100

f32 = jnp.float32
i32 = jnp.int32

_MESH = dict(core_axis_name="c", subcore_axis_name="s")


def _zero_rows(ref, nrows, d):
    def fill(i, _):
        for q in range(d // 16):
            ref[i, pl.ds(q * 16, 16)] = jnp.zeros((16,), f32)
        return 0
    lax.fori_loop(0, nrows, fill, 0)


@functools.partial(
    pl.kernel,
    mesh=plsc.VectorSubcoreMesh(**_MESH),
    compiler_params=pltpu.CompilerParams(use_tc_tiling_on_sc=False),
    out_type=jax.ShapeDtypeStruct((NC, NPAD, 16), f32),
    scratch_types=[
        pltpu.VMEM((NCH, CHUNK), i32),
        pltpu.VMEM((CHUNK, 16), f32),
        pltpu.VMEM((RPT, 16), f32),
        pltpu.VMEM_SHARED((NPAD, 16), f32),
    ],
)
def _sc_degree(dst_hbm, out_hbm, dst_v, ones_v, zbuf, acc):
    c = lax.axis_index("c")
    s = lax.axis_index("s")
    wid = c * NS + s
    pltpu.sync_copy(dst_hbm.at[wid], dst_v)
    _zero_rows(zbuf, RPT, 16)

    def fill1(i, _):
        ones_v[i, :] = jnp.ones((16,), f32)
        return 0
    lax.fori_loop(0, CHUNK, fill1, 0)

    pltpu.sync_copy(zbuf, acc.at[pl.ds(s * RPT, RPT)])
    plsc.subcore_barrier()

    def body(j, _):
        pltpu.sync_copy(ones_v, acc.at[dst_v.at[j]], add=True)
        return 0
    lax.fori_loop(0, NCH, body, 0)

    plsc.subcore_barrier()
    pltpu.sync_copy(acc.at[pl.ds(s * RPT, RPT)],
                    out_hbm.at[c, pl.ds(s * RPT, RPT)])


def _make_sc_scatter(d):
    @functools.partial(
        pl.kernel,
        mesh=plsc.VectorSubcoreMesh(**_MESH),
        compiler_params=pltpu.CompilerParams(use_tc_tiling_on_sc=False),
        out_type=jax.ShapeDtypeStruct((NC, NPAD, d), f32),
        scratch_types=[
            pltpu.VMEM((NCH, CHUNK), i32),
            pltpu.VMEM((NCH, CHUNK), i32),
            pltpu.VMEM((CHUNK, d), f32),
            pltpu.VMEM((CHUNK, d), f32),
            pltpu.VMEM((RPT, d), f32),
            pltpu.VMEM_SHARED((NPAD, d), f32),
            pltpu.SemaphoreType.DMA,
            pltpu.SemaphoreType.DMA,
        ],
    )
    def sc_scatter(table_hbm, src_hbm, dst_hbm, out_hbm,
                   src_v, dst_v, ra, rb, zbuf, acc, sa, sb):
        c = lax.axis_index("c")
        s = lax.axis_index("s")
        wid = c * NS + s
        pltpu.sync_copy(src_hbm.at[wid], src_v)
        pltpu.sync_copy(dst_hbm.at[wid], dst_v)
        _zero_rows(zbuf, RPT, d)
        pltpu.sync_copy(zbuf, acc.at[pl.ds(s * RPT, RPT)])
        plsc.subcore_barrier()

        pltpu.make_async_copy(table_hbm.at[src_v.at[0]], ra, sa).start()

        def body(i, _):
            j0 = 2 * i
            pltpu.make_async_copy(table_hbm.at[src_v.at[j0 + 1]], rb, sb).start()
            pltpu.make_async_copy(table_hbm.at[src_v.at[j0]], ra, sa).wait()
            pltpu.sync_copy(ra, acc.at[dst_v.at[j0]], add=True)

            @pl.when(i < NCH // 2 - 1)
            def _():
                pltpu.make_async_copy(table_hbm.at[src_v.at[j0 + 2]], ra, sa).start()

            pltpu.make_async_copy(table_hbm.at[src_v.at[j0 + 1]], rb, sb).wait()
            pltpu.sync_copy(rb, acc.at[dst_v.at[j0 + 1]], add=True)
            return 0
        lax.fori_loop(0, NCH // 2, body, 0)

        plsc.subcore_barrier()
        pltpu.sync_copy(acc.at[pl.ds(s * RPT, RPT)],
                        out_hbm.at[c, pl.ds(s * RPT, RPT)])
    return sc_scatter


_sc_scatter16 = _make_sc_scatter(16)
_sc_scatter32 = _make_sc_scatter(32)


_M = 512
_G = NPAD // _M


def _tc_pre(state2d, degp, w1p):
    def body(x_ref, dp_ref, w_ref, hp_ref, dinv_ref, ta_ref):
        i = pl.program_id(0)
        deg = dp_ref[0, :, :1] + dp_ref[1, :, :1] + 1.0
        dinv = lax.rsqrt(deg)
        x = x_ref[...]
        hp_ref[...] = jnp.dot(x, w_ref[...], preferred_element_type=f32) * dinv
        dinv_ref[...] = jnp.broadcast_to(dinv, (_M, 16))
        part = jnp.sum(x[:, 1:2]).reshape(1, 1)

        @pl.when(i == 0)
        def _():
            ta_ref[...] = part

        @pl.when(i != 0)
        def _():
            ta_ref[...] = ta_ref[...] + part

    return pl.pallas_call(
        body,
        grid=(_G,),
        in_specs=[
            pl.BlockSpec((_M, 128), lambda i: (i, 0)),
            pl.BlockSpec((NC, _M, 16), lambda i: (0, i, 0)),
            pl.BlockSpec((128, 16), lambda i: (0, 0)),
        ],
        out_specs=[
            pl.BlockSpec((_M, 16), lambda i: (i, 0)),
            pl.BlockSpec((_M, 16), lambda i: (i, 0)),
            pl.BlockSpec((1, 1), lambda i: (0, 0)),
        ],
        out_shape=[
            jax.ShapeDtypeStruct((NPAD, 16), f32),
            jax.ShapeDtypeStruct((NPAD, 16), f32),
            jax.ShapeDtypeStruct((1, 1), f32),
        ],
    )(state2d, degp, w1p)


def _make_tc_conv(d_in, d_out):
    def run(spart, hp, dinv, wnext, b):
        def body(s_ref, hp_ref, dinv_ref, w_ref, b_ref, out_ref, hn_ref):
            dinv = dinv_ref[:, :1]
            o = jnp.maximum(
                dinv * (s_ref[0] + s_ref[1] + hp_ref[...]) + b_ref[...], 0.0)
            out_ref[...] = o
            hn_ref[...] = jnp.dot(o, w_ref[...], preferred_element_type=f32) * dinv

        return pl.pallas_call(
            body,
            grid=(_G,),
            in_specs=[
                pl.BlockSpec((NC, _M, d_in), lambda i: (0, i, 0)),
                pl.BlockSpec((_M, d_in), lambda i: (i, 0)),
                pl.BlockSpec((_M, 16), lambda i: (i, 0)),
                pl.BlockSpec((d_in, d_out), lambda i: (0, 0)),
                pl.BlockSpec((1, d_in), lambda i: (0, 0)),
            ],
            out_specs=[
                pl.BlockSpec((_M, d_in), lambda i: (i, 0)),
                pl.BlockSpec((_M, d_out), lambda i: (i, 0)),
            ],
            out_shape=[
                jax.ShapeDtypeStruct((NPAD, d_in), f32),
                jax.ShapeDtypeStruct((NPAD, d_out), f32),
            ],
        )(spart, hp, dinv, wnext, b)
    return run


_tc_conv1 = _make_tc_conv(16, 32)
_tc_conv2 = _make_tc_conv(32, 32)


def _leaky(z):
    return jnp.where(z > 0, z, 0.01 * z)


def _tc_head(spart, hp, dinv, b3, o1, o2, o3, o4, state2d, ta,
             l1a, l1b, l1c, l1d, l1e, l1f, l1g, lb1, lw2, lb2, lw3p, lb3):
    def body(s_ref, hp_ref, dinv_ref, b3_ref, o1_ref, o2_ref, o3_ref, o4_ref,
             x_ref, ta_ref, a_ref, b_ref, c_ref, d_ref, e_ref, f_ref, g_ref,
             lb1_ref, lw2_ref, lb2_ref, lw3_ref, lb3_ref, conc_ref, tot_ref):
        i = pl.program_id(0)
        dinv = dinv_ref[:, :1]
        o5 = jnp.maximum(
            dinv * (s_ref[0] + s_ref[1] + hp_ref[...]) + b3_ref[...], 0.0)
        dot = lambda a, b: jnp.dot(a, b, preferred_element_type=f32)
        acc = (dot(o1_ref[...], a_ref[...]) + dot(o2_ref[...], b_ref[...])
               + dot(o3_ref[...], c_ref[...]) + dot(o4_ref[...], d_ref[...])
               + dot(o5, e_ref[...]) + dot(x_ref[...], f_ref[...])
               + ta_ref[...] * g_ref[...] + lb1_ref[...])
        x1 = _leaky(acc)
        x2 = _leaky(dot(x1, lw2_ref[...]) + lb2_ref[...])
        z = dot(x2, lw3_ref[...])[:, :1] + lb3_ref[...]
        conc = jnp.maximum(z, 0.0) + jnp.log1p(jnp.exp(-jnp.abs(z)))
        ridx = i * _M + lax.broadcasted_iota(i32, (_M, 1), 0)
        conc = jnp.where(ridx < N, conc, 0.0)
        conc_ref[...] = conc
        part = jnp.sum(conc).reshape(1, 1)

        @pl.when(i == 0)
        def _():
            tot_ref[...] = part

        @pl.when(i != 0)
        def _():
            tot_ref[...] = tot_ref[...] + part

    return pl.pallas_call(
        body,
        grid=(_G,),
        in_specs=[
            pl.BlockSpec((NC, _M, 32), lambda i: (0, i, 0)),
            pl.BlockSpec((_M, 32), lambda i: (i, 0)),
            pl.BlockSpec((_M, 16), lambda i: (i, 0)),
            pl.BlockSpec((1, 32), lambda i: (0, 0)),
            pl.BlockSpec((_M, 16), lambda i: (i, 0)),
            pl.BlockSpec((_M, 32), lambda i: (i, 0)),
            pl.BlockSpec((_M, 32), lambda i: (i, 0)),
            pl.BlockSpec((_M, 32), lambda i: (i, 0)),
            pl.BlockSpec((_M, 128), lambda i: (i, 0)),
            pl.BlockSpec((1, 1), lambda i: (0, 0)),
            pl.BlockSpec((16, 32), lambda i: (0, 0)),
            pl.BlockSpec((32, 32), lambda i: (0, 0)),
            pl.BlockSpec((32, 32), lambda i: (0, 0)),
            pl.BlockSpec((32, 32), lambda i: (0, 0)),
            pl.BlockSpec((32, 32), lambda i: (0, 0)),
            pl.BlockSpec((128, 32), lambda i: (0, 0)),
            pl.BlockSpec((1, 32), lambda i: (0, 0)),
            pl.BlockSpec((1, 32), lambda i: (0, 0)),
            pl.BlockSpec((32, 32), lambda i: (0, 0)),
            pl.BlockSpec((1, 32), lambda i: (0, 0)),
            pl.BlockSpec((32, 128), lambda i: (0, 0)),
            pl.BlockSpec((1, 1), lambda i: (0, 0)),
        ],
        out_specs=[
            pl.BlockSpec((_M, 1), lambda i: (i, 0)),
            pl.BlockSpec((1, 1), lambda i: (0, 0)),
        ],
        out_shape=[
            jax.ShapeDtypeStruct((NPAD, 1), f32),
            jax.ShapeDtypeStruct((1, 1), f32),
        ],
    )(spart, hp, dinv, b3, o1, o2, o3, o4, state2d, ta,
      l1a, l1b, l1c, l1d, l1e, l1f, l1g, lb1, lw2, lb2, lw3p, lb3)


def _tc_norm(conc, tot):
    def body(c_ref, t_ref, act_ref, reg_ref):
        t = t_ref[0, 0]
        act_ref[...] = c_ref[...] / (t + 1e-20)
        reg_ref[...] = (t / N).reshape(1, 1)

    return pl.pallas_call(
        body,
        grid=(1,),
        in_specs=[
            pl.BlockSpec((NPAD, 1), lambda i: (0, 0)),
            pl.BlockSpec((1, 1), lambda i: (0, 0)),
        ],
        out_specs=[
            pl.BlockSpec((NPAD, 1), lambda i: (0, 0)),
            pl.BlockSpec((1, 1), lambda i: (0, 0)),
        ],
        out_shape=[
            jax.ShapeDtypeStruct((NPAD, 1), f32),
            jax.ShapeDtypeStruct((1, 1), f32),
        ],
    )(conc, tot)


def _pad2(a, rows, cols):
    return jnp.pad(a, ((0, rows - a.shape[0]), (0, cols - a.shape[1])))


def kernel(state, edge_index, deterministic,
           W1, b1, W2, b2, W3, b3, lw1, lb1, lw2, lb2, lw3, lb3):
    ei = edge_index.astype(i32)
    epw = E // NW
    pad = jnp.full((NW, EPW - epw), PAD_NODE, i32)
    srcp = jnp.concatenate([ei[0].reshape(NW, epw), pad], 1).reshape(NW, NCH, CHUNK)
    dstp = jnp.concatenate([ei[1].reshape(NW, epw), pad], 1).reshape(NW, NCH, CHUNK)

    state2d = jnp.pad(state[0], ((0, NPAD - N), (0, 0)))

    w1p = _pad2(W1, 128, 16)
    w2p = _pad2(W2, 16, 32)
    w3p = _pad2(W3, 32, 32)
    b1p = _pad2(b1[None, :], 1, 16)
    b2p = _pad2(b2[None, :], 1, 32)
    b3p = _pad2(b3[None, :], 1, 32)
    l1a = _pad2(lw1[0:10], 16, 32)
    l1b = _pad2(lw1[10:31], 32, 32)
    l1c = _pad2(lw1[31:52], 32, 32)
    l1d = _pad2(lw1[52:73], 32, 32)
    l1e = _pad2(lw1[73:94], 32, 32)
    l1f = lw1[94:222]
    l1g = lw1[222:223]
    lb1p = lb1[None, :]
    lb2p = lb2[None, :]
    lw3p = _pad2(lw3, 32, 128)
    lb3p = lb3[None, :]

    degp = _sc_degree(dstp)
    h1p, dinv, ta = _tc_pre(state2d, degp, w1p)

    s1 = _sc_scatter16(h1p, srcp, dstp)
    o1, h2p = _tc_conv1(s1, h1p, dinv, w2p, b1p)

    s2 = _sc_scatter32(h2p, srcp, dstp)
    o2, h3p = _tc_conv2(s2, h2p, dinv, w3p, b2p)

    s3 = _sc_scatter32(h3p, srcp, dstp)
    o3, h4p = _tc_conv2(s3, h3p, dinv, w3p, b3p)

    s4 = _sc_scatter32(h4p, srcp, dstp)
    o4, h5p = _tc_conv2(s4, h4p, dinv, w3p, b3p)

    s5 = _sc_scatter32(h5p, srcp, dstp)
    conc, tot = _tc_head(s5, h5p, dinv, b3p, o1, o2, o3, o4, state2d, ta,
                         l1a, l1b, l1c, l1d, l1e, l1f, l1g,
                         lb1p, lw2, lb2p, lw3p, lb3p)

    act, reg = _tc_norm(conc, tot)
    action = act[:N, 0][None, :]
    regularize = reg[0, 0]
    return (action, regularize)


# speedup vs baseline: 21.4315x; 1.1275x over previous
import functools

import jax
import jax.numpy as jnp
from jax import lax
from jax.experimental import pallas as pl
from jax.experimental.pallas import tpu as pltpu
from jax.experimental.pallas import tpu_sc as plsc

N = 10000
NPAD = 10240
E = 320000
NC = 2
NS = 16
NW = NC * NS
EPW = 10240
CHUNK = 128
NCH = EPW // CHUNK
RPT = NPAD // NS
PAD_NODE = N + 100

f32 = jnp.float32
i32 = jnp.int32

_MESH = dict(core_axis_name="c", subcore_axis_name="s")


def _zero_rows(ref, nrows, d):
    def fill(i, _):
        for q in range(d // 16):
            ref[i, pl.ds(q * 16, 16)] = jnp.zeros((16,), f32)
        return 0
    lax.fori_loop(0, nrows, fill, 0)


_KD = 16


@functools.partial(
    pl.kernel,
    mesh=plsc.VectorSubcoreMesh(**_MESH),
    compiler_params=pltpu.CompilerParams(use_tc_tiling_on_sc=False),
    out_type=jax.ShapeDtypeStruct((NC, NPAD, 16), f32),
    scratch_types=[
        pltpu.VMEM((NCH, CHUNK), i32),
        pltpu.VMEM((CHUNK, 16), f32),
        pltpu.VMEM((RPT, 16), f32),
        pltpu.VMEM_SHARED((NPAD, 16), f32),
        pltpu.SemaphoreType.DMA,
    ],
)
def _sc_degree(dst_hbm, out_hbm, dst_v, ones_v, zbuf, acc, ssem):
    c = lax.axis_index("c")
    s = lax.axis_index("s")
    wid = c * NS + s
    pltpu.sync_copy(dst_hbm.at[wid], dst_v)
    _zero_rows(zbuf, RPT, 16)

    def fill1(i, _):
        ones_v[i, :] = jnp.ones((16,), f32)
        return 0
    lax.fori_loop(0, CHUNK, fill1, 0)

    pltpu.sync_copy(zbuf, acc.at[pl.ds(s * RPT, RPT)])
    plsc.subcore_barrier()

    def body(t, _):
        base = _KD * t
        for k in range(_KD):
            pltpu.make_async_copy(
                ones_v, acc.at[dst_v.at[base + k]], ssem).start()
        for k in range(_KD):
            pltpu.make_async_copy(
                ones_v, acc.at[dst_v.at[base + k]], ssem).wait()
        return 0
    lax.fori_loop(0, NCH // _KD, body, 0)

    plsc.subcore_barrier()
    pltpu.sync_copy(acc.at[pl.ds(s * RPT, RPT)],
                    out_hbm.at[c, pl.ds(s * RPT, RPT)])


_K = 8


def _make_sc_scatter(d):
    @functools.partial(
        pl.kernel,
        mesh=plsc.VectorSubcoreMesh(**_MESH),
        compiler_params=pltpu.CompilerParams(use_tc_tiling_on_sc=False),
        out_type=jax.ShapeDtypeStruct((NC, NPAD, d), f32),
        scratch_types=[
            pltpu.VMEM((NCH, CHUNK), i32),
            pltpu.VMEM((NCH, CHUNK), i32),
            pltpu.VMEM((_K, CHUNK, d), f32),
            pltpu.VMEM((RPT, d), f32),
            pltpu.VMEM_SHARED((NPAD, d), f32),
            pltpu.SemaphoreType.DMA((_K,)),
            pltpu.SemaphoreType.DMA((_K,)),
        ],
    )
    def sc_scatter(table_hbm, src_hbm, dst_hbm, out_hbm,
                   src_v, dst_v, rows, zbuf, acc, gsem, ssem):
        c = lax.axis_index("c")
        s = lax.axis_index("s")
        wid = c * NS + s
        pltpu.sync_copy(src_hbm.at[wid], src_v)
        pltpu.sync_copy(dst_hbm.at[wid], dst_v)
        _zero_rows(zbuf, RPT, d)
        pltpu.sync_copy(zbuf, acc.at[pl.ds(s * RPT, RPT)])
        plsc.subcore_barrier()

        for k in range(_K):
            pltpu.make_async_copy(
                table_hbm.at[src_v.at[k]], rows.at[k], gsem.at[k]).start()

        def body(t, _):
            base = _K * t
            for k in range(_K):
                pltpu.make_async_copy(
                    table_hbm.at[src_v.at[base + k]], rows.at[k],
                    gsem.at[k]).wait()
                pltpu.make_async_copy(
                    rows.at[k], acc.at[dst_v.at[base + k]], ssem.at[k]).start()
            for k in range(_K):
                pltpu.make_async_copy(
                    rows.at[k], acc.at[dst_v.at[base + k]], ssem.at[k]).wait()

                @pl.when(t < NCH // _K - 1)
                def _():
                    pltpu.make_async_copy(
                        table_hbm.at[src_v.at[base + _K + k]], rows.at[k],
                        gsem.at[k]).start()
            return 0
        lax.fori_loop(0, NCH // _K, body, 0)

        plsc.subcore_barrier()
        pltpu.sync_copy(acc.at[pl.ds(s * RPT, RPT)],
                        out_hbm.at[c, pl.ds(s * RPT, RPT)])
    return sc_scatter


_sc_scatter16 = _make_sc_scatter(16)
_sc_scatter32 = _make_sc_scatter(32)


_M = 512
_G = NPAD // _M


def _tc_pre(state2d, degp, w1p):
    def body(x_ref, dp_ref, w_ref, hp_ref, dinv_ref, ta_ref):
        i = pl.program_id(0)
        deg = dp_ref[0, :, :1] + dp_ref[1, :, :1] + 1.0
        dinv = lax.rsqrt(deg)
        x = x_ref[...]
        hp_ref[...] = jnp.dot(x, w_ref[...], preferred_element_type=f32) * dinv
        dinv_ref[...] = jnp.broadcast_to(dinv, (_M, 16))
        part = jnp.sum(x[:, 1:2]).reshape(1, 1)

        @pl.when(i == 0)
        def _():
            ta_ref[...] = part

        @pl.when(i != 0)
        def _():
            ta_ref[...] = ta_ref[...] + part

    return pl.pallas_call(
        body,
        grid=(_G,),
        in_specs=[
            pl.BlockSpec((_M, 128), lambda i: (i, 0)),
            pl.BlockSpec((NC, _M, 16), lambda i: (0, i, 0)),
            pl.BlockSpec((128, 16), lambda i: (0, 0)),
        ],
        out_specs=[
            pl.BlockSpec((_M, 16), lambda i: (i, 0)),
            pl.BlockSpec((_M, 16), lambda i: (i, 0)),
            pl.BlockSpec((1, 1), lambda i: (0, 0)),
        ],
        out_shape=[
            jax.ShapeDtypeStruct((NPAD, 16), f32),
            jax.ShapeDtypeStruct((NPAD, 16), f32),
            jax.ShapeDtypeStruct((1, 1), f32),
        ],
    )(state2d, degp, w1p)


def _make_tc_conv(d_in, d_out):
    def run(spart, hp, dinv, wnext, b):
        def body(s_ref, hp_ref, dinv_ref, w_ref, b_ref, out_ref, hn_ref):
            dinv = dinv_ref[:, :1]
            o = jnp.maximum(
                dinv * (s_ref[0] + s_ref[1] + hp_ref[...]) + b_ref[...], 0.0)
            out_ref[...] = o
            hn_ref[...] = jnp.dot(o, w_ref[...], preferred_element_type=f32) * dinv

        return pl.pallas_call(
            body,
            grid=(_G,),
            in_specs=[
                pl.BlockSpec((NC, _M, d_in), lambda i: (0, i, 0)),
                pl.BlockSpec((_M, d_in), lambda i: (i, 0)),
                pl.BlockSpec((_M, 16), lambda i: (i, 0)),
                pl.BlockSpec((d_in, d_out), lambda i: (0, 0)),
                pl.BlockSpec((1, d_in), lambda i: (0, 0)),
            ],
            out_specs=[
                pl.BlockSpec((_M, d_in), lambda i: (i, 0)),
                pl.BlockSpec((_M, d_out), lambda i: (i, 0)),
            ],
            out_shape=[
                jax.ShapeDtypeStruct((NPAD, d_in), f32),
                jax.ShapeDtypeStruct((NPAD, d_out), f32),
            ],
        )(spart, hp, dinv, wnext, b)
    return run


_tc_conv1 = _make_tc_conv(16, 32)
_tc_conv2 = _make_tc_conv(32, 32)


def _leaky(z):
    return jnp.where(z > 0, z, 0.01 * z)


def _tc_head(spart, hp, dinv, b3, o1, o2, o3, o4, state2d, ta,
             l1a, l1b, l1c, l1d, l1e, l1f, l1g, lb1, lw2, lb2, lw3p, lb3):
    def body(s_ref, hp_ref, dinv_ref, b3_ref, o1_ref, o2_ref, o3_ref, o4_ref,
             x_ref, ta_ref, a_ref, b_ref, c_ref, d_ref, e_ref, f_ref, g_ref,
             lb1_ref, lw2_ref, lb2_ref, lw3_ref, lb3_ref, conc_ref, tot_ref):
        i = pl.program_id(0)
        dinv = dinv_ref[:, :1]
        o5 = jnp.maximum(
            dinv * (s_ref[0] + s_ref[1] + hp_ref[...]) + b3_ref[...], 0.0)
        dot = lambda a, b: jnp.dot(a, b, preferred_element_type=f32)
        acc = (dot(o1_ref[...], a_ref[...]) + dot(o2_ref[...], b_ref[...])
               + dot(o3_ref[...], c_ref[...]) + dot(o4_ref[...], d_ref[...])
               + dot(o5, e_ref[...]) + dot(x_ref[...], f_ref[...])
               + ta_ref[...] * g_ref[...] + lb1_ref[...])
        x1 = _leaky(acc)
        x2 = _leaky(dot(x1, lw2_ref[...]) + lb2_ref[...])
        z = dot(x2, lw3_ref[...])[:, :1] + lb3_ref[...]
        conc = jnp.maximum(z, 0.0) + jnp.log1p(jnp.exp(-jnp.abs(z)))
        ridx = i * _M + lax.broadcasted_iota(i32, (_M, 1), 0)
        conc = jnp.where(ridx < N, conc, 0.0)
        conc_ref[...] = conc
        part = jnp.sum(conc).reshape(1, 1)

        @pl.when(i == 0)
        def _():
            tot_ref[...] = part

        @pl.when(i != 0)
        def _():
            tot_ref[...] = tot_ref[...] + part

    return pl.pallas_call(
        body,
        grid=(_G,),
        in_specs=[
            pl.BlockSpec((NC, _M, 32), lambda i: (0, i, 0)),
            pl.BlockSpec((_M, 32), lambda i: (i, 0)),
            pl.BlockSpec((_M, 16), lambda i: (i, 0)),
            pl.BlockSpec((1, 32), lambda i: (0, 0)),
            pl.BlockSpec((_M, 16), lambda i: (i, 0)),
            pl.BlockSpec((_M, 32), lambda i: (i, 0)),
            pl.BlockSpec((_M, 32), lambda i: (i, 0)),
            pl.BlockSpec((_M, 32), lambda i: (i, 0)),
            pl.BlockSpec((_M, 128), lambda i: (i, 0)),
            pl.BlockSpec((1, 1), lambda i: (0, 0)),
            pl.BlockSpec((16, 32), lambda i: (0, 0)),
            pl.BlockSpec((32, 32), lambda i: (0, 0)),
            pl.BlockSpec((32, 32), lambda i: (0, 0)),
            pl.BlockSpec((32, 32), lambda i: (0, 0)),
            pl.BlockSpec((32, 32), lambda i: (0, 0)),
            pl.BlockSpec((128, 32), lambda i: (0, 0)),
            pl.BlockSpec((1, 32), lambda i: (0, 0)),
            pl.BlockSpec((1, 32), lambda i: (0, 0)),
            pl.BlockSpec((32, 32), lambda i: (0, 0)),
            pl.BlockSpec((1, 32), lambda i: (0, 0)),
            pl.BlockSpec((32, 128), lambda i: (0, 0)),
            pl.BlockSpec((1, 1), lambda i: (0, 0)),
        ],
        out_specs=[
            pl.BlockSpec((_M, 1), lambda i: (i, 0)),
            pl.BlockSpec((1, 1), lambda i: (0, 0)),
        ],
        out_shape=[
            jax.ShapeDtypeStruct((NPAD, 1), f32),
            jax.ShapeDtypeStruct((1, 1), f32),
        ],
    )(spart, hp, dinv, b3, o1, o2, o3, o4, state2d, ta,
      l1a, l1b, l1c, l1d, l1e, l1f, l1g, lb1, lw2, lb2, lw3p, lb3)


def _tc_norm(conc, tot):
    def body(c_ref, t_ref, act_ref, reg_ref):
        t = t_ref[0, 0]
        act_ref[...] = c_ref[...] / (t + 1e-20)
        reg_ref[...] = (t / N).reshape(1, 1)

    return pl.pallas_call(
        body,
        grid=(1,),
        in_specs=[
            pl.BlockSpec((NPAD, 1), lambda i: (0, 0)),
            pl.BlockSpec((1, 1), lambda i: (0, 0)),
        ],
        out_specs=[
            pl.BlockSpec((NPAD, 1), lambda i: (0, 0)),
            pl.BlockSpec((1, 1), lambda i: (0, 0)),
        ],
        out_shape=[
            jax.ShapeDtypeStruct((NPAD, 1), f32),
            jax.ShapeDtypeStruct((1, 1), f32),
        ],
    )(conc, tot)


def _pad2(a, rows, cols):
    return jnp.pad(a, ((0, rows - a.shape[0]), (0, cols - a.shape[1])))


def kernel(state, edge_index, deterministic,
           W1, b1, W2, b2, W3, b3, lw1, lb1, lw2, lb2, lw3, lb3):
    ei = edge_index.astype(i32)
    epw = E // NW
    pad = jnp.full((NW, EPW - epw), PAD_NODE, i32)
    srcp = jnp.concatenate([ei[0].reshape(NW, epw), pad], 1).reshape(NW, NCH, CHUNK)
    dstp = jnp.concatenate([ei[1].reshape(NW, epw), pad], 1).reshape(NW, NCH, CHUNK)

    state2d = jnp.pad(state[0], ((0, NPAD - N), (0, 0)))

    w1p = _pad2(W1, 128, 16)
    w2p = _pad2(W2, 16, 32)
    w3p = _pad2(W3, 32, 32)
    b1p = _pad2(b1[None, :], 1, 16)
    b2p = _pad2(b2[None, :], 1, 32)
    b3p = _pad2(b3[None, :], 1, 32)
    l1a = _pad2(lw1[0:10], 16, 32)
    l1b = _pad2(lw1[10:31], 32, 32)
    l1c = _pad2(lw1[31:52], 32, 32)
    l1d = _pad2(lw1[52:73], 32, 32)
    l1e = _pad2(lw1[73:94], 32, 32)
    l1f = lw1[94:222]
    l1g = lw1[222:223]
    lb1p = lb1[None, :]
    lb2p = lb2[None, :]
    lw3p = _pad2(lw3, 32, 128)
    lb3p = lb3[None, :]

    degp = _sc_degree(dstp)
    h1p, dinv, ta = _tc_pre(state2d, degp, w1p)

    s1 = _sc_scatter16(h1p, srcp, dstp)
    o1, h2p = _tc_conv1(s1, h1p, dinv, w2p, b1p)

    s2 = _sc_scatter32(h2p, srcp, dstp)
    o2, h3p = _tc_conv2(s2, h2p, dinv, w3p, b2p)

    s3 = _sc_scatter32(h3p, srcp, dstp)
    o3, h4p = _tc_conv2(s3, h3p, dinv, w3p, b3p)

    s4 = _sc_scatter32(h4p, srcp, dstp)
    o4, h5p = _tc_conv2(s4, h4p, dinv, w3p, b3p)

    s5 = _sc_scatter32(h5p, srcp, dstp)
    conc, tot = _tc_head(s5, h5p, dinv, b3p, o1, o2, o3, o4, state2d, ta,
                         l1a, l1b, l1c, l1d, l1e, l1f, l1g,
                         lb1p, lw2, lb2p, lw3p, lb3p)

    act, reg = _tc_norm(conc, tot)
    action = act[:N, 0][None, :]
    regularize = reg[0, 0]
    return (action, regularize)


# speedup vs baseline: 24.0691x; 1.1231x over previous
import functools

import jax
import jax.numpy as jnp
from jax import lax
from jax.experimental import pallas as pl
from jax.experimental.pallas import tpu as pltpu
from jax.experimental.pallas import tpu_sc as plsc

N = 10000
NPAD = 10240
E = 320000
NC = 2
NS = 16
NW = NC * NS
EPW = 10240
CHUNK = 128
NCH = EPW // CHUNK
RPT = NPAD // NS
PAD_NODE = N + 100

f32 = jnp.float32
i32 = jnp.int32

_MESH = dict(core_axis_name="c", subcore_axis_name="s")


_KD = 16


@functools.partial(
    pl.kernel,
    mesh=plsc.VectorSubcoreMesh(**_MESH),
    compiler_params=pltpu.CompilerParams(use_tc_tiling_on_sc=False),
    out_type=jax.ShapeDtypeStruct((NC, NPAD, 16), f32),
    scratch_types=[
        pltpu.VMEM((NCH, CHUNK), i32),
        pltpu.VMEM((CHUNK, 16), f32),
        pltpu.VMEM_SHARED((NPAD, 16), f32),
        pltpu.SemaphoreType.DMA,
    ],
)
def _sc_degree(dst_hbm, ones_hbm, zeros_hbm, out_hbm, dst_v, ones_v, acc, ssem):
    c = lax.axis_index("c")
    s = lax.axis_index("s")
    wid = c * NS + s
    pltpu.sync_copy(dst_hbm.at[wid], dst_v)
    pltpu.sync_copy(ones_hbm, ones_v)
    pltpu.sync_copy(zeros_hbm.at[pl.ds(0, RPT)], acc.at[pl.ds(s * RPT, RPT)])
    plsc.subcore_barrier()

    def body(t, _):
        base = _KD * t
        for k in range(_KD):
            pltpu.make_async_copy(
                ones_v, acc.at[dst_v.at[base + k]], ssem).start()
        for k in range(_KD):
            pltpu.make_async_copy(
                ones_v, acc.at[dst_v.at[base + k]], ssem).wait()
        return 0
    lax.fori_loop(0, NCH // _KD, body, 0)

    plsc.subcore_barrier()
    pltpu.sync_copy(acc.at[pl.ds(s * RPT, RPT)],
                    out_hbm.at[c, pl.ds(s * RPT, RPT)])


_K = 8


def _make_sc_scatter(d):
    @functools.partial(
        pl.kernel,
        mesh=plsc.VectorSubcoreMesh(**_MESH),
        compiler_params=pltpu.CompilerParams(use_tc_tiling_on_sc=False),
        out_type=jax.ShapeDtypeStruct((NC, NPAD, d), f32),
        scratch_types=[
            pltpu.VMEM((NCH, CHUNK), i32),
            pltpu.VMEM((NCH, CHUNK), i32),
            pltpu.VMEM((_K, CHUNK, d), f32),
            pltpu.VMEM_SHARED((NPAD, d), f32),
            pltpu.SemaphoreType.DMA((_K,)),
            pltpu.SemaphoreType.DMA((_K,)),
        ],
    )
    def sc_scatter(table_hbm, src_hbm, dst_hbm, zeros_hbm, out_hbm,
                   src_v, dst_v, rows, acc, gsem, ssem):
        c = lax.axis_index("c")
        s = lax.axis_index("s")
        wid = c * NS + s
        pltpu.sync_copy(src_hbm.at[wid], src_v)
        pltpu.sync_copy(dst_hbm.at[wid], dst_v)
        pltpu.sync_copy(zeros_hbm.at[pl.ds(0, RPT)], acc.at[pl.ds(s * RPT, RPT)])
        plsc.subcore_barrier()

        for k in range(_K):
            pltpu.make_async_copy(
                table_hbm.at[src_v.at[k]], rows.at[k], gsem.at[k]).start()

        def body(t, _):
            base = _K * t
            for k in range(_K):
                pltpu.make_async_copy(
                    table_hbm.at[src_v.at[base + k]], rows.at[k],
                    gsem.at[k]).wait()
                pltpu.make_async_copy(
                    rows.at[k], acc.at[dst_v.at[base + k]], ssem.at[k]).start()
            for k in range(_K):
                pltpu.make_async_copy(
                    rows.at[k], acc.at[dst_v.at[base + k]], ssem.at[k]).wait()

                @pl.when(t < NCH // _K - 1)
                def _():
                    pltpu.make_async_copy(
                        table_hbm.at[src_v.at[base + _K + k]], rows.at[k],
                        gsem.at[k]).start()
            return 0
        lax.fori_loop(0, NCH // _K, body, 0)

        plsc.subcore_barrier()
        pltpu.sync_copy(acc.at[pl.ds(s * RPT, RPT)],
                        out_hbm.at[c, pl.ds(s * RPT, RPT)])
    return sc_scatter


_sc_scatter16 = _make_sc_scatter(16)
_sc_scatter24 = _make_sc_scatter(24)


_M = 512
_G = NPAD // _M


def _tc_pre(state2d, degp, w1p):
    def body(x_ref, dp_ref, w_ref, hp_ref, dinv_ref, ta_ref):
        i = pl.program_id(0)
        deg = dp_ref[0, :, :1] + dp_ref[1, :, :1] + 1.0
        dinv = lax.rsqrt(deg)
        x = x_ref[...]
        hp_ref[...] = jnp.dot(x, w_ref[...], preferred_element_type=f32) * dinv
        dinv_ref[...] = jnp.broadcast_to(dinv, (_M, 16))
        part = jnp.sum(x[:, 1:2]).reshape(1, 1)

        @pl.when(i == 0)
        def _():
            ta_ref[...] = part

        @pl.when(i != 0)
        def _():
            ta_ref[...] = ta_ref[...] + part

    return pl.pallas_call(
        body,
        grid=(_G,),
        in_specs=[
            pl.BlockSpec((_M, 128), lambda i: (i, 0)),
            pl.BlockSpec((NC, _M, 16), lambda i: (0, i, 0)),
            pl.BlockSpec((128, 16), lambda i: (0, 0)),
        ],
        out_specs=[
            pl.BlockSpec((_M, 16), lambda i: (i, 0)),
            pl.BlockSpec((_M, 16), lambda i: (i, 0)),
            pl.BlockSpec((1, 1), lambda i: (0, 0)),
        ],
        out_shape=[
            jax.ShapeDtypeStruct((NPAD, 16), f32),
            jax.ShapeDtypeStruct((NPAD, 16), f32),
            jax.ShapeDtypeStruct((1, 1), f32),
        ],
    )(state2d, degp, w1p)


def _make_tc_conv(d_in, d_out):
    def run(spart, hp, dinv, wnext, b):
        def body(s_ref, hp_ref, dinv_ref, w_ref, b_ref, out_ref, hn_ref):
            dinv = dinv_ref[:, :1]
            o = jnp.maximum(
                dinv * (s_ref[0] + s_ref[1] + hp_ref[...]) + b_ref[...], 0.0)
            out_ref[...] = o
            hn_ref[...] = jnp.dot(o, w_ref[...], preferred_element_type=f32) * dinv

        return pl.pallas_call(
            body,
            grid=(_G,),
            in_specs=[
                pl.BlockSpec((NC, _M, d_in), lambda i: (0, i, 0)),
                pl.BlockSpec((_M, d_in), lambda i: (i, 0)),
                pl.BlockSpec((_M, 16), lambda i: (i, 0)),
                pl.BlockSpec((d_in, d_out), lambda i: (0, 0)),
                pl.BlockSpec((1, d_in), lambda i: (0, 0)),
            ],
            out_specs=[
                pl.BlockSpec((_M, d_in), lambda i: (i, 0)),
                pl.BlockSpec((_M, d_out), lambda i: (i, 0)),
            ],
            out_shape=[
                jax.ShapeDtypeStruct((NPAD, d_in), f32),
                jax.ShapeDtypeStruct((NPAD, d_out), f32),
            ],
        )(spart, hp, dinv, wnext, b)
    return run


_tc_conv1 = _make_tc_conv(16, 24)
_tc_conv2 = _make_tc_conv(24, 24)


def _leaky(z):
    return jnp.where(z > 0, z, 0.01 * z)


def _tc_head(spart, hp, dinv, b3, o1, o2, o3, o4, state2d, ta,
             l1a, l1b, l1c, l1d, l1e, l1f, l1g, lb1, lw2, lb2, lw3p, lb3):
    def body(s_ref, hp_ref, dinv_ref, b3_ref, o1_ref, o2_ref, o3_ref, o4_ref,
             x_ref, ta_ref, a_ref, b_ref, c_ref, d_ref, e_ref, f_ref, g_ref,
             lb1_ref, lw2_ref, lb2_ref, lw3_ref, lb3_ref, conc_ref, tot_ref):
        i = pl.program_id(0)
        dinv = dinv_ref[:, :1]
        o5 = jnp.maximum(
            dinv * (s_ref[0] + s_ref[1] + hp_ref[...]) + b3_ref[...], 0.0)
        dot = lambda a, b: jnp.dot(a, b, preferred_element_type=f32)
        acc = (dot(o1_ref[...], a_ref[...]) + dot(o2_ref[...], b_ref[...])
               + dot(o3_ref[...], c_ref[...]) + dot(o4_ref[...], d_ref[...])
               + dot(o5, e_ref[...]) + dot(x_ref[...], f_ref[...])
               + ta_ref[...] * g_ref[...] + lb1_ref[...])
        x1 = _leaky(acc)
        x2 = _leaky(dot(x1, lw2_ref[...]) + lb2_ref[...])
        z = dot(x2, lw3_ref[...])[:, :1] + lb3_ref[...]
        conc = jnp.maximum(z, 0.0) + jnp.log1p(jnp.exp(-jnp.abs(z)))
        ridx = i * _M + lax.broadcasted_iota(i32, (_M, 1), 0)
        conc = jnp.where(ridx < N, conc, 0.0)
        conc_ref[...] = conc
        part = jnp.sum(conc).reshape(1, 1)

        @pl.when(i == 0)
        def _():
            tot_ref[...] = part

        @pl.when(i != 0)
        def _():
            tot_ref[...] = tot_ref[...] + part

    return pl.pallas_call(
        body,
        grid=(_G,),
        in_specs=[
            pl.BlockSpec((NC, _M, 24), lambda i: (0, i, 0)),
            pl.BlockSpec((_M, 24), lambda i: (i, 0)),
            pl.BlockSpec((_M, 16), lambda i: (i, 0)),
            pl.BlockSpec((1, 24), lambda i: (0, 0)),
            pl.BlockSpec((_M, 16), lambda i: (i, 0)),
            pl.BlockSpec((_M, 24), lambda i: (i, 0)),
            pl.BlockSpec((_M, 24), lambda i: (i, 0)),
            pl.BlockSpec((_M, 24), lambda i: (i, 0)),
            pl.BlockSpec((_M, 128), lambda i: (i, 0)),
            pl.BlockSpec((1, 1), lambda i: (0, 0)),
            pl.BlockSpec((16, 32), lambda i: (0, 0)),
            pl.BlockSpec((24, 32), lambda i: (0, 0)),
            pl.BlockSpec((24, 32), lambda i: (0, 0)),
            pl.BlockSpec((24, 32), lambda i: (0, 0)),
            pl.BlockSpec((24, 32), lambda i: (0, 0)),
            pl.BlockSpec((128, 32), lambda i: (0, 0)),
            pl.BlockSpec((1, 32), lambda i: (0, 0)),
            pl.BlockSpec((1, 32), lambda i: (0, 0)),
            pl.BlockSpec((32, 32), lambda i: (0, 0)),
            pl.BlockSpec((1, 32), lambda i: (0, 0)),
            pl.BlockSpec((32, 128), lambda i: (0, 0)),
            pl.BlockSpec((1, 1), lambda i: (0, 0)),
        ],
        out_specs=[
            pl.BlockSpec((_M, 1), lambda i: (i, 0)),
            pl.BlockSpec((1, 1), lambda i: (0, 0)),
        ],
        out_shape=[
            jax.ShapeDtypeStruct((NPAD, 1), f32),
            jax.ShapeDtypeStruct((1, 1), f32),
        ],
    )(spart, hp, dinv, b3, o1, o2, o3, o4, state2d, ta,
      l1a, l1b, l1c, l1d, l1e, l1f, l1g, lb1, lw2, lb2, lw3p, lb3)


def _tc_norm(conc, tot):
    def body(c_ref, t_ref, act_ref, reg_ref):
        t = t_ref[0, 0]
        act_ref[...] = c_ref[...] / (t + 1e-20)
        reg_ref[...] = (t / N).reshape(1, 1)

    return pl.pallas_call(
        body,
        grid=(1,),
        in_specs=[
            pl.BlockSpec((NPAD, 1), lambda i: (0, 0)),
            pl.BlockSpec((1, 1), lambda i: (0, 0)),
        ],
        out_specs=[
            pl.BlockSpec((NPAD, 1), lambda i: (0, 0)),
            pl.BlockSpec((1, 1), lambda i: (0, 0)),
        ],
        out_shape=[
            jax.ShapeDtypeStruct((NPAD, 1), f32),
            jax.ShapeDtypeStruct((1, 1), f32),
        ],
    )(conc, tot)


def _pad2(a, rows, cols):
    return jnp.pad(a, ((0, rows - a.shape[0]), (0, cols - a.shape[1])))


def kernel(state, edge_index, deterministic,
           W1, b1, W2, b2, W3, b3, lw1, lb1, lw2, lb2, lw3, lb3):
    ei = edge_index.astype(i32)
    epw = E // NW
    pad = jnp.full((NW, EPW - epw), PAD_NODE, i32)
    srcp = jnp.concatenate([ei[0].reshape(NW, epw), pad], 1).reshape(NW, NCH, CHUNK)
    dstp = jnp.concatenate([ei[1].reshape(NW, epw), pad], 1).reshape(NW, NCH, CHUNK)

    state2d = jnp.pad(state[0], ((0, NPAD - N), (0, 0)))

    w1p = _pad2(W1, 128, 16)
    w2p = _pad2(W2, 16, 24)
    w3p = _pad2(W3, 24, 24)
    b1p = _pad2(b1[None, :], 1, 16)
    b2p = _pad2(b2[None, :], 1, 24)
    b3p = _pad2(b3[None, :], 1, 24)
    l1a = _pad2(lw1[0:10], 16, 32)
    l1b = _pad2(lw1[10:31], 24, 32)
    l1c = _pad2(lw1[31:52], 24, 32)
    l1d = _pad2(lw1[52:73], 24, 32)
    l1e = _pad2(lw1[73:94], 24, 32)
    l1f = lw1[94:222]
    l1g = lw1[222:223]
    lb1p = lb1[None, :]
    lb2p = lb2[None, :]
    lw3p = _pad2(lw3, 32, 128)
    lb3p = lb3[None, :]

    ones16 = jnp.ones((CHUNK, 16), f32)
    z16 = jnp.zeros((RPT, 16), f32)
    z24 = jnp.zeros((RPT, 24), f32)

    degp = _sc_degree(dstp, ones16, z16)
    h1p, dinv, ta = _tc_pre(state2d, degp, w1p)

    s1 = _sc_scatter16(h1p, srcp, dstp, z16)
    o1, h2p = _tc_conv1(s1, h1p, dinv, w2p, b1p)

    s2 = _sc_scatter24(h2p, srcp, dstp, z24)
    o2, h3p = _tc_conv2(s2, h2p, dinv, w3p, b2p)

    s3 = _sc_scatter24(h3p, srcp, dstp, z24)
    o3, h4p = _tc_conv2(s3, h3p, dinv, w3p, b3p)

    s4 = _sc_scatter24(h4p, srcp, dstp, z24)
    o4, h5p = _tc_conv2(s4, h4p, dinv, w3p, b3p)

    s5 = _sc_scatter24(h5p, srcp, dstp, z24)
    conc, tot = _tc_head(s5, h5p, dinv, b3p, o1, o2, o3, o4, state2d, ta,
                         l1a, l1b, l1c, l1d, l1e, l1f, l1g,
                         lb1p, lw2, lb2p, lw3p, lb3p)

    act, reg = _tc_norm(conc, tot)
    action = act[:N, 0][None, :]
    regularize = reg[0, 0]
    return (action, regularize)
